# Initial kernel scaffold; baseline (speedup 1.0000x reference)
import jax
import jax.numpy as jnp
from jax.experimental import pallas as pl


def kernel(memory_features, memory_labels, batch_features, batch_labels, ptr):
    raise NotImplementedError("write your pallas kernel here")


# speedup vs baseline: 2.5683x; 2.5683x over previous
import jax
import jax.numpy as jnp
from jax import lax
from jax.experimental import pallas as pl
from jax.experimental.pallas import tpu as pltpu
from jax.experimental.pallas import tpu_sc as plsc

M = 100000
D = 128
B = 16384
NC = 2
NS = 16
NW = NC * NS
RPW = B // NW
G = 8
NG = RPW // G
LW = 25
LS = M // LW
LSTEPS = LS // 16


def _body(feat_hbm, ml_hbm, bf_hbm, bl_hbm, ptr_hbm, mlout_hbm,
          feat_v, lab_v, bl_v, ptr_v):
    cid = lax.axis_index("c")
    sid = lax.axis_index("s")
    wid = cid * NS + sid

    pltpu.sync_copy(ptr_hbm, ptr_v)
    p = ptr_v[...][0]
    T = M - p

    base = wid * RPW
    pltpu.sync_copy(bf_hbm.at[pl.ds(base, RPW)], feat_v)
    tail_all = base + RPW <= T
    head_all = base >= T

    @pl.when(tail_all)
    def _():
        pltpu.sync_copy(feat_v, feat_hbm.at[pl.ds(p + base, RPW)])

    @pl.when(head_all)
    def _():
        pltpu.sync_copy(feat_v, feat_hbm.at[pl.ds(base - T, RPW)])

    @pl.when(jnp.logical_not(jnp.logical_or(tail_all, head_all)))
    def _():
        def gbody(i, carry):
            j = base + i * G
            straddle = jnp.logical_and(j < T, T < j + G)

            @pl.when(jnp.logical_not(straddle))
            def _():
                d = jnp.where(j >= T, j - T, p + j)
                pltpu.sync_copy(feat_v.at[pl.ds(i * G, G)],
                                feat_hbm.at[pl.ds(d, G)])

            @pl.when(straddle)
            def _():
                for r in range(G):
                    jr = j + r
                    dr = jnp.where(jr >= T, jr - T, p + jr)
                    pltpu.sync_copy(feat_v.at[pl.ds(i * G + r, 1)],
                                    feat_hbm.at[pl.ds(dr, 1)])

            return carry
        lax.fori_loop(0, NG, gbody, 0)

    @pl.when(wid < LW)
    def _():
        s0 = wid * LS
        pltpu.sync_copy(ml_hbm.at[pl.ds(s0, LS)], lab_v)
        pltpu.sync_copy(bl_hbm, bl_v)
        lanes = lax.iota(jnp.int32, 16)

        def lbody(i, carry):
            off = i * 16
            g = s0 + off + lanes
            t1 = g - p
            j = jnp.where(t1 < 0, t1 + M, t1)
            mask = j < B
            jc = jnp.where(mask, j, 0)
            gathered = plsc.load_gather(bl_v, [jc])
            cur = lab_v[pl.ds(off, 16)]
            lab_v[pl.ds(off, 16)] = jnp.where(mask, gathered, cur)
            return carry
        lax.fori_loop(0, LSTEPS, lbody, 0)
        pltpu.sync_copy(lab_v, mlout_hbm.at[pl.ds(s0, LS)])


_scatter = pl.kernel(
    _body,
    out_type=jax.ShapeDtypeStruct((M,), jnp.int32),
    mesh=plsc.VectorSubcoreMesh(core_axis_name="c", subcore_axis_name="s",
                                num_cores=NC, num_subcores=NS),
    compiler_params=pltpu.CompilerParams(use_tc_tiling_on_sc=False,
                                         needs_layout_passes=False),
    scratch_types=[
        pltpu.VMEM((RPW, D), jnp.float32),
        pltpu.VMEM((LS,), jnp.int32),
        pltpu.VMEM((B,), jnp.int32),
        pltpu.VMEM((16,), jnp.int32),
    ],
)


def kernel(memory_features, memory_labels, batch_features, batch_labels, ptr):
    ptr32 = jnp.asarray(ptr, jnp.int32)
    ptr_arr = jnp.full((16,), ptr32, dtype=jnp.int32)
    feat_ref = jax.new_ref(memory_features)
    new_labels = _scatter(feat_ref, memory_labels, batch_features,
                          batch_labels, ptr_arr)
    new_features = feat_ref[...]
    new_ptr = (ptr32 + B) % M
    return new_features, new_labels, new_ptr
